# baseline (speedup 1.0000x reference)
import functools
import numpy as np
import jax
import jax.numpy as jnp
from jax import lax
from jax.experimental import pallas as pl
from jax.experimental.pallas import tpu as pltpu
from jax.experimental.pallas import tpu_sc as plsc

IN_DIM_K = 8192
OUT_DIM_K = 16
GRID_K = 16

_KNOTS = np.linspace(-1.0, 1.0, GRID_K).astype(np.float32)
_INV = (1.0 / (_KNOTS[1:] - _KNOTS[:-1] + np.float32(1e-8))).astype(np.float32)


def _table_kernel(x_ref, v_ref, mw_ref, b_ref, a_ref, idx_ref):
    xc = jnp.clip(x_ref[...], -1.0, 1.0)

    idxf = jnp.full_like(xc, 1.0)
    x0 = jnp.full_like(xc, _KNOTS[0])
    invd = jnp.full_like(xc, _INV[0])
    for g in range(1, GRID_K - 1):
        c = (xc > _KNOTS[g]).astype(jnp.float32)
        idxf = idxf + c
        x0 = x0 + c * (_KNOTS[g] - _KNOTS[g - 1])
        invd = invd + c * (_INV[g] - _INV[g - 1])
    t = (xc - x0) * invd

    v = v_ref[...]
    mw = mw_ref[...]
    wt = mw * t
    mv = jnp.dot(mw, v, preferred_element_type=jnp.float32)
    mt = jnp.dot(wt, v, preferred_element_type=jnp.float32)

    ao_hi = mv[:, : GRID_K - 1] + mt[:, 1:] - mt[:, : GRID_K - 1]
    ao = jnp.concatenate([jnp.zeros((OUT_DIM_K, 1), jnp.float32), ao_hi], axis=1)
    a_ref[...] = ao.T + b_ref[...]
    idx_ref[...] = idxf.astype(jnp.int32)


def _make_sc_gather():
    info = plsc.get_sparse_core_info()
    nc, ns, nl = info.num_cores, info.num_subcores, info.num_lanes
    nw = nc * ns
    rows_per_w = IN_DIM_K // nw
    n_chunks = rows_per_w // 128
    mesh = plsc.VectorSubcoreMesh(core_axis_name="c", subcore_axis_name="s")

    flat_per_w = rows_per_w * OUT_DIM_K

    @functools.partial(
        pl.kernel,
        mesh=mesh,
        compiler_params=pltpu.CompilerParams(needs_layout_passes=False),
        out_type=jax.ShapeDtypeStruct((IN_DIM_K * OUT_DIM_K,), jnp.float32),
        scratch_types=[
            pltpu.VMEM((GRID_K * OUT_DIM_K,), jnp.float32),
            pltpu.VMEM((n_chunks, 128), jnp.int32),
            pltpu.VMEM((flat_per_w,), jnp.float32),
            pltpu.SemaphoreType.DMA,
        ],
    )
    def sc_gather(a_hbm, idx_hbm, out_hbm, a_v, idx_v, out_v, sem):
        wid = lax.axis_index("s") * nc + lax.axis_index("c")
        cp_a = pltpu.async_copy(a_hbm, a_v, sem)
        cp_i = pltpu.async_copy(idx_hbm.at[pl.ds(wid * n_chunks, n_chunks)], idx_v, sem)
        cp_a.wait()
        cp_i.wait()
        lane16 = lax.iota(jnp.int32, nl) * OUT_DIM_K
        for c in range(n_chunks):
            for g in range(128 // nl):
                idxvec = idx_v[c, pl.ds(g * nl, nl)]
                src = idxvec * OUT_DIM_K
                dst = lane16 + (c * 128 + g * nl) * OUT_DIM_K
                for o in range(OUT_DIM_K):
                    vals = plsc.load_gather(a_v, [src + o])
                    plsc.store_scatter(out_v, [dst + o], vals)
        pltpu.sync_copy(out_v, out_hbm.at[pl.ds(wid * flat_per_w, flat_per_w)])

    return sc_gather


_sc_gather = _make_sc_gather()


def kernel(x, values, mix_w, mix_b):
    xr = x.reshape(1, IN_DIM_K)
    br = mix_b.reshape(1, OUT_DIM_K)
    a, idx = pl.pallas_call(
        _table_kernel,
        out_shape=[
            jax.ShapeDtypeStruct((GRID_K, OUT_DIM_K), jnp.float32),
            jax.ShapeDtypeStruct((1, IN_DIM_K), jnp.int32),
        ],
    )(xr, values, mix_w, br)
    out_flat = _sc_gather(a.reshape(GRID_K * OUT_DIM_K), idx.reshape(64, 128))
    return out_flat.reshape(IN_DIM_K, OUT_DIM_K)


# speedup vs baseline: 1.0041x; 1.0041x over previous
import functools
import numpy as np
import jax
import jax.numpy as jnp
from jax import lax
from jax.experimental import pallas as pl
from jax.experimental.pallas import tpu as pltpu
from jax.experimental.pallas import tpu_sc as plsc

IN_DIM_K = 8192
OUT_DIM_K = 16
GRID_K = 16

_KNOTS = np.linspace(-1.0, 1.0, GRID_K).astype(np.float32)
_INV = (1.0 / (_KNOTS[1:] - _KNOTS[:-1] + np.float32(1e-8))).astype(np.float32)


def _table_kernel(x_ref, v_ref, mw_ref, b_ref, a_ref, idx_ref):
    xc = jnp.clip(x_ref[...], -1.0, 1.0)

    idxf = jnp.full_like(xc, 1.0)
    x0 = jnp.full_like(xc, _KNOTS[0])
    invd = jnp.full_like(xc, _INV[0])
    for g in range(1, GRID_K - 1):
        c = (xc > _KNOTS[g]).astype(jnp.float32)
        idxf = idxf + c
        x0 = x0 + c * (_KNOTS[g] - _KNOTS[g - 1])
        invd = invd + c * (_INV[g] - _INV[g - 1])
    t = (xc - x0) * invd

    v = v_ref[...]
    mw = mw_ref[...]
    wt = mw * t
    mv = jnp.dot(mw, v, preferred_element_type=jnp.float32)
    mt = jnp.dot(wt, v, preferred_element_type=jnp.float32)

    ao_hi = mv[:, : GRID_K - 1] + mt[:, 1:] - mt[:, : GRID_K - 1]
    ao = jnp.concatenate([jnp.zeros((OUT_DIM_K, 1), jnp.float32), ao_hi], axis=1)
    a_ref[...] = ao.T + b_ref[...]
    idx_ref[...] = idxf.astype(jnp.int32)


def _make_sc_gather():
    info = plsc.get_sparse_core_info()
    nc, ns, nl = info.num_cores, info.num_subcores, info.num_lanes
    nw = nc * ns
    rows_per_w = IN_DIM_K // nw
    n_chunks = rows_per_w // 128
    mesh = plsc.VectorSubcoreMesh(core_axis_name="c", subcore_axis_name="s")

    flat_per_w = rows_per_w * OUT_DIM_K

    @functools.partial(
        pl.kernel,
        mesh=mesh,
        compiler_params=pltpu.CompilerParams(needs_layout_passes=False),
        out_type=jax.ShapeDtypeStruct((IN_DIM_K * OUT_DIM_K,), jnp.float32),
        scratch_types=[
            pltpu.VMEM((GRID_K * OUT_DIM_K,), jnp.float32),
            pltpu.VMEM((rows_per_w,), jnp.int32),
            pltpu.VMEM((flat_per_w,), jnp.float32),
            pltpu.SemaphoreType.DMA,
        ],
    )
    def sc_gather(a_hbm, idx_hbm, out_hbm, a_v, idx_v, out_v, sem):
        wid = lax.axis_index("s") * nc + lax.axis_index("c")
        cp_a = pltpu.async_copy(a_hbm, a_v, sem)
        cp_i = pltpu.async_copy(
            idx_hbm.at[0, pl.ds(wid * rows_per_w, rows_per_w)], idx_v, sem
        )
        cp_a.wait()
        cp_i.wait()
        lane16 = lax.iota(jnp.int32, nl) * OUT_DIM_K
        for g in range(rows_per_w // nl):
            idxvec = idx_v[pl.ds(g * nl, nl)]
            src = idxvec * OUT_DIM_K
            dst = lane16 + (g * nl) * OUT_DIM_K
            for o in range(OUT_DIM_K):
                vals = plsc.load_gather(a_v, [src + o])
                plsc.store_scatter(out_v, [dst + o], vals)
        pltpu.sync_copy(out_v, out_hbm.at[pl.ds(wid * flat_per_w, flat_per_w)])

    return sc_gather


_sc_gather = _make_sc_gather()


def kernel(x, values, mix_w, mix_b):
    xr = x.reshape(1, IN_DIM_K)
    br = mix_b.reshape(1, OUT_DIM_K)
    a, idx = pl.pallas_call(
        _table_kernel,
        out_shape=[
            jax.ShapeDtypeStruct((GRID_K, OUT_DIM_K), jnp.float32),
            jax.ShapeDtypeStruct((1, IN_DIM_K), jnp.int32),
        ],
    )(xr, values, mix_w, br)
    out_flat = _sc_gather(a.reshape(GRID_K * OUT_DIM_K), idx)
    return out_flat.reshape(IN_DIM_K, OUT_DIM_K)
